# Initial kernel scaffold; baseline (speedup 1.0000x reference)
import functools

import jax
import jax.numpy as jnp
from jax import lax
from jax.experimental import pallas as pl
from jax.experimental.pallas import tpu as pltpu
from jax.experimental.pallas import tpu_sc as plsc

B, T, D, H = 16, 2048, 128, 64
_NC = 2
_LANES = 16


def _sc_body(time_a, time_b, len_a, len_b, feat_a, feat_b,
             out_rows, out_g, ta_v, tb_v, la_v, lb_v, row_v, g_v):
    wid = lax.axis_index("s") * _NC + lax.axis_index("c")

    @pl.when(wid < B)
    def _():
        b = wid
        pltpu.sync_copy(time_a.at[b], ta_v)
        pltpu.sync_copy(time_b.at[b], tb_v)
        pltpu.sync_copy(len_a, la_v)
        pltpu.sync_copy(len_b, lb_v)
        la = la_v[b]
        lb = lb_v[b]
        lane = lax.iota(jnp.int32, _LANES)
        neg = jnp.float32(-1.0)

        def make_step(tv, lim, off):
            def body(i, carry):
                vmax, vidx = carry
                v = tv[pl.ds(i * _LANES, _LANES)]
                posv = lane + i * _LANES
                mv = jnp.where(posv < lim, v, neg)
                take = mv >= vmax
                return (jnp.where(take, mv, vmax),
                        jnp.where(take, posv + off, vidx))
            return body

        init = (jnp.full((_LANES,), neg, jnp.float32),
                jnp.zeros((_LANES,), jnp.int32))
        carry = lax.fori_loop(0, T // _LANES, make_step(ta_v, la, 0), init)
        vmax, vidx = lax.fori_loop(0, T // _LANES, make_step(tb_v, lb, T), carry)

        m = jnp.max(vmax)
        cand = jnp.where(vmax == m, vidx, jnp.int32(-1))
        j = jnp.max(cand)
        flag = j >= T
        p = jnp.where(flag, j - T, j)

        @pl.when(flag)
        def _():
            pltpu.sync_copy(feat_b.at[b, p], row_v)

        @pl.when(jnp.logical_not(flag))
        def _():
            pltpu.sync_copy(feat_a.at[b, p], row_v)

        gval = jnp.where(flag, jnp.float32(1.0), jnp.float32(0.0))
        gvec = jnp.zeros((_LANES,), jnp.float32) + gval
        for k in range(128 // _LANES):
            g_v[pl.ds(k * _LANES, _LANES)] = gvec
        pltpu.sync_copy(row_v, out_rows.at[b])
        pltpu.sync_copy(g_v, out_g.at[b])


_sc_select = functools.partial(
    pl.kernel,
    out_type=(jax.ShapeDtypeStruct((B, D), jnp.float32),
              jax.ShapeDtypeStruct((B, 128), jnp.float32)),
    mesh=plsc.VectorSubcoreMesh(core_axis_name="c", subcore_axis_name="s"),
    scratch_types=[
        pltpu.VMEM((T,), jnp.float32),
        pltpu.VMEM((T,), jnp.float32),
        pltpu.VMEM((B,), jnp.int32),
        pltpu.VMEM((B,), jnp.int32),
        pltpu.VMEM((D,), jnp.float32),
        pltpu.VMEM((128,), jnp.float32),
    ],
)()(_sc_body)


def _tc_body(rows_ref, g_ref, wa_ref, ba_ref, wb_ref, bb_ref, ws_ref, bs_ref,
             out_ref):
    f = rows_ref[...]
    g = g_ref[...][:, :H]
    ea = jnp.dot(f, wa_ref[...], preferred_element_type=jnp.float32) + ba_ref[...]
    eb = jnp.dot(f, wb_ref[...], preferred_element_type=jnp.float32) + bb_ref[...]
    h = ea + g * (eb - ea)
    out_ref[...] = (jnp.dot(h, ws_ref[...], preferred_element_type=jnp.float32)
                    + bs_ref[...])


_tc_encode = pl.pallas_call(
    _tc_body,
    out_shape=jax.ShapeDtypeStruct((B, H), jnp.float32),
)


def kernel(time_a, feat_a, len_a, time_b, feat_b, len_b,
           W_a, b_a, W_b, b_b, W_seq, b_seq):
    la = len_a.astype(jnp.int32)
    lb = len_b.astype(jnp.int32)
    rows, g = _sc_select(time_a, time_b, la, lb, feat_a, feat_b)
    return _tc_encode(rows, g, W_a, b_a.reshape(1, H), W_b, b_b.reshape(1, H),
                      W_seq, b_seq.reshape(1, H))


# speedup vs baseline: 53.4851x; 53.4851x over previous
import functools

import jax
import jax.numpy as jnp
from jax import lax
from jax.experimental import pallas as pl
from jax.experimental.pallas import tpu as pltpu
from jax.experimental.pallas import tpu_sc as plsc

B, T, D, H = 16, 2048, 128, 64
_NC = 2
_LANES = 16


def _sc_body(time_a, time_b, len_a, len_b, feat_a, feat_b,
             out_rows, out_g, ta_v, tb_v, la_v, lb_v, row_v, g_v):
    wid = lax.axis_index("s") * _NC + lax.axis_index("c")

    @pl.when(wid < B)
    def _():
        b = wid
        pltpu.sync_copy(time_a.at[b], ta_v)
        pltpu.sync_copy(time_b.at[b], tb_v)
        pltpu.sync_copy(len_a, la_v)
        pltpu.sync_copy(len_b, lb_v)
        lane = lax.iota(jnp.int32, _LANES)
        sel = lane == b
        la = jnp.max(jnp.where(sel, la_v[...], jnp.int32(0)))
        lb = jnp.max(jnp.where(sel, lb_v[...], jnp.int32(0)))
        neg = jnp.float32(-1.0)

        def make_step(tv, lim, off):
            def body(i, carry):
                vmax, vidx = carry
                v = tv[pl.ds(i * _LANES, _LANES)]
                posv = lane + i * _LANES
                mv = jnp.where(posv < lim, v, neg)
                take = mv >= vmax
                return (jnp.where(take, mv, vmax),
                        jnp.where(take, posv + off, vidx))
            return body

        init = (jnp.full((_LANES,), neg, jnp.float32),
                jnp.zeros((_LANES,), jnp.int32))
        carry = lax.fori_loop(0, T // _LANES, make_step(ta_v, la, 0), init)
        vmax, vidx = lax.fori_loop(0, T // _LANES, make_step(tb_v, lb, T), carry)

        m = jnp.max(vmax)
        cand = jnp.where(vmax == m, vidx, jnp.int32(-1))
        j = jnp.max(cand)
        flag = j >= T
        p = jnp.where(flag, j - T, j)

        @pl.when(flag)
        def _():
            pltpu.sync_copy(feat_b.at[b, p], row_v)

        @pl.when(jnp.logical_not(flag))
        def _():
            pltpu.sync_copy(feat_a.at[b, p], row_v)

        gval = jnp.where(flag, jnp.float32(1.0), jnp.float32(0.0))
        gvec = jnp.zeros((_LANES,), jnp.float32) + gval
        for k in range(128 // _LANES):
            g_v[pl.ds(k * _LANES, _LANES)] = gvec
        pltpu.sync_copy(row_v, out_rows.at[b])
        pltpu.sync_copy(g_v, out_g.at[b])


_sc_select = pl.kernel(
    _sc_body,
    out_type=(jax.ShapeDtypeStruct((B, D), jnp.float32),
              jax.ShapeDtypeStruct((B, 128), jnp.float32)),
    mesh=plsc.VectorSubcoreMesh(core_axis_name="c", subcore_axis_name="s"),
    compiler_params=pltpu.CompilerParams(needs_layout_passes=False),
    scratch_types=[
        pltpu.VMEM((T,), jnp.float32),
        pltpu.VMEM((T,), jnp.float32),
        pltpu.VMEM((B,), jnp.int32),
        pltpu.VMEM((B,), jnp.int32),
        pltpu.VMEM((D,), jnp.float32),
        pltpu.VMEM((128,), jnp.float32),
    ],
)


def _tc_body(rows_ref, g_ref, wa_ref, ba_ref, wb_ref, bb_ref, ws_ref, bs_ref,
             out_ref):
    f = rows_ref[...]
    g = g_ref[...][:, :H]
    ea = jnp.dot(f, wa_ref[...], preferred_element_type=jnp.float32) + ba_ref[...]
    eb = jnp.dot(f, wb_ref[...], preferred_element_type=jnp.float32) + bb_ref[...]
    h = ea + g * (eb - ea)
    out_ref[...] = (jnp.dot(h, ws_ref[...], preferred_element_type=jnp.float32)
                    + bs_ref[...])


_tc_encode = pl.pallas_call(
    _tc_body,
    out_shape=jax.ShapeDtypeStruct((B, H), jnp.float32),
)


def kernel(time_a, feat_a, len_a, time_b, feat_b, len_b,
           W_a, b_a, W_b, b_b, W_seq, b_seq):
    la = len_a.astype(jnp.int32)
    lb = len_b.astype(jnp.int32)
    rows, g = _sc_select(time_a, time_b, la, lb, feat_a, feat_b)
    return _tc_encode(rows, g, W_a, b_a.reshape(1, H), W_b, b_b.reshape(1, H),
                      W_seq, b_seq.reshape(1, H))
